# baseline (speedup 1.0000x reference)
import functools

import jax
import jax.numpy as jnp
from jax import lax
from jax.experimental import pallas as pl
from jax.experimental.pallas import tpu as pltpu
from jax.experimental.pallas import tpu_sc as plsc

_CB = 8192
_D = 32
_TOK_BLK = 1024
_CB_CHUNK = 2048
_BETA = 0.25


def _vq_body(x_ref, cb_ref, ind_ref, loss_ref):
    xt = x_ref[0]
    x16 = xt.astype(jnp.bfloat16)
    sq = xt * xt
    sx = sq[0:1, :]
    for k in range(1, _D):
        sx = sx + sq[k:k + 1, :]
    iota = jax.lax.broadcasted_iota(
        jnp.int32, (_CB_CHUNK, _TOK_BLK), 0)

    accv = jnp.full((1, _TOK_BLK), -jnp.inf, dtype=jnp.float32)
    acci = jnp.zeros((1, _TOK_BLK), dtype=jnp.int32)
    accd = jnp.zeros((1, _TOK_BLK), dtype=jnp.float32)
    for c0 in range(0, _CB, _CB_CHUNK):
        cb = cb_ref[c0:c0 + _CB_CHUNK, :]
        sc = jnp.sum(cb * cb, axis=1, keepdims=True)
        mm = jax.lax.dot_general(
            cb, x16, dimension_numbers=(((1,), (0,)), ((), ())),
            preferred_element_type=jnp.float32)
        d = (-sx - sc) + 2.0 * mm
        m = jnp.max(d, axis=0, keepdims=True)
        li = jnp.min(jnp.where(d == m, iota, _CB), axis=0, keepdims=True)
        upd = m > accv
        acci = jnp.where(upd, c0 + li, acci)
        accd = jnp.where(upd, m, accd)
        accv = jnp.where(upd, m, accv).astype(jnp.bfloat16).astype(jnp.float32)

    ind_ref[0, :, :] = acci
    loss_ref[0, 0, :] = jnp.broadcast_to(jnp.sum(accd), (128,))


def _tc_distance_argmax(x3d, codebook):
    n_blk = x3d.shape[0]
    n_tok = n_blk * x3d.shape[2]
    ind, loss_parts = pl.pallas_call(
        _vq_body,
        grid=(n_blk,),
        in_specs=[
            pl.BlockSpec((1, _D, _TOK_BLK), lambda i: (i, 0, 0)),
            pl.BlockSpec((_CB, _D), lambda i: (0, 0)),
        ],
        out_specs=[
            pl.BlockSpec((1, 1, _TOK_BLK), lambda i: (i, 0, 0)),
            pl.BlockSpec((1, 1, 128), lambda i: (i, 0, 0)),
        ],
        out_shape=[
            jax.ShapeDtypeStruct((n_blk, 1, _TOK_BLK), jnp.int32),
            jax.ShapeDtypeStruct((n_blk, 1, 128), jnp.float32),
        ],
        compiler_params=pltpu.CompilerParams(
            dimension_semantics=("arbitrary",)),
    )(x3d, codebook)
    return ind.reshape(n_tok), loss_parts


def _sc_gather_st(codebook, idx, x3d):
    b, d, hw = x3d.shape
    n_tok = b * hw
    mesh = plsc.VectorSubcoreMesh(core_axis_name="c", subcore_axis_name="s")
    info = plsc.get_sparse_core_info()
    nw = info.num_cores * info.num_subcores
    t_per_w = n_tok // nw
    w_per_b = hw // t_per_w

    @functools.partial(
        pl.kernel, mesh=mesh,
        out_type=jax.ShapeDtypeStruct((n_tok, d), jnp.float32),
        scratch_types=[
            pltpu.VMEM((t_per_w,), jnp.int32),
            pltpu.VMEM((t_per_w, d), jnp.float32),
            pltpu.SemaphoreType.DMA,
        ],
        compiler_params=pltpu.CompilerParams(use_tc_tiling_on_sc=False),
    )
    def gather_kernel(table_hbm, idx_hbm, out_hbm, idx_v, rows_v, sem):
        wid = lax.axis_index("s") * info.num_cores + lax.axis_index("c")
        base = wid * t_per_w
        pltpu.sync_copy(idx_hbm.at[pl.ds(base, t_per_w)], idx_v)
        pltpu.async_copy(table_hbm.at[idx_v], rows_v, sem).wait()
        pltpu.sync_copy(rows_v, out_hbm.at[pl.ds(base, t_per_w)])

    return gather_kernel(codebook, idx)


def kernel(x, codebook):
    b, c, h, w = x.shape
    n_tok = b * h * w
    hw_ = h * w
    x3d = x.reshape(b, c, hw_)

    ind, loss_parts = _tc_distance_argmax(x3d, codebook)
    xq_rows = _sc_gather_st(codebook, ind, x3d)

    embed_ind = ind.reshape(b, h, w)
    loss = (-(1.0 + _BETA) / (n_tok * c)) * jnp.sum(loss_parts[:, 0, 0])
    xqT = jnp.transpose(xq_rows.reshape(b, hw_, c), (0, 2, 1))
    x_q = (x3d + (xqT - x3d)).reshape(b, c, h, w)
    return (x_q, loss, embed_ind)
